# Initial kernel scaffold; baseline (speedup 1.0000x reference)
import jax
import jax.numpy as jnp
from jax.experimental import pallas as pl


def kernel(yhat, y, b, s):
    raise NotImplementedError("write your pallas kernel here")


# speedup vs baseline: 2.5483x; 2.5483x over previous
import jax
import jax.numpy as jnp
from jax.experimental import pallas as pl
from jax.experimental.pallas import tpu as pltpu

_EPS = 0.1
_N = 16384
_T = 512
_K = 64
_BR = 512
_G = _N // _BR


def _nse_kernel(b_ref, s_ref, yhat_ref, y_ref, out_ref, acc_ref):
    i = pl.program_id(0)

    @pl.when(i == 0)
    def _init():
        acc_ref[...] = jnp.zeros_like(acc_ref)

    d = yhat_ref[...] - y_ref[...]
    d2 = d * d
    b_row = b_ref[...].reshape(1, _BR)
    kio = jax.lax.broadcasted_iota(jnp.int32, (_K, _BR), 0)
    m = (kio == b_row).astype(jnp.float32)
    acc_ref[...] += jnp.dot(m, d2, preferred_element_type=jnp.float32)

    @pl.when(i == _G - 1)
    def _fin():
        wtab = 1.0 / (s_ref[...] + _EPS) ** 2
        tot = jnp.sum(wtab * acc_ref[...]) * (1.0 / (_N * _T))
        out_ref[...] = tot.reshape(1, 1)


def kernel(yhat, y, b, s):
    b3 = b.astype(jnp.int32).reshape(_G, 1, _BR)
    s2 = s.reshape(_K, 1)
    out = pl.pallas_call(
        _nse_kernel,
        grid=(_G,),
        in_specs=[
            pl.BlockSpec((1, 1, _BR), lambda i: (i, 0, 0)),
            pl.BlockSpec((_K, 1), lambda i: (0, 0)),
            pl.BlockSpec((_BR, _T), lambda i: (i, 0)),
            pl.BlockSpec((_BR, _T), lambda i: (i, 0)),
        ],
        out_specs=pl.BlockSpec((1, 1), lambda i: (0, 0)),
        out_shape=jax.ShapeDtypeStruct((1, 1), jnp.float32),
        scratch_shapes=[pltpu.VMEM((_K, _T), jnp.float32)],
        compiler_params=pltpu.CompilerParams(
            dimension_semantics=("arbitrary",),
        ),
    )(b3, s2, yhat, y)
    return out[0, 0]


# speedup vs baseline: 3.3416x; 1.3113x over previous
import jax
import jax.numpy as jnp
from jax.experimental import pallas as pl
from jax.experimental.pallas import tpu as pltpu

_EPS = 0.1
_N = 16384
_T = 512
_K = 64
_BR = 1024
_G = _N // _BR


def _nse_kernel(b_ref, s_ref, yhat_ref, y_ref, out_ref, acc_ref):
    i = pl.program_id(0)

    @pl.when(i == 0)
    def _init():
        acc_ref[...] = jnp.zeros_like(acc_ref)

    d = yhat_ref[...] - y_ref[...]
    d2 = (d * d).astype(jnp.bfloat16)
    b_row = b_ref[...].reshape(1, _BR)
    kio = jax.lax.broadcasted_iota(jnp.int32, (_K, _BR), 0)
    m = (kio == b_row).astype(jnp.bfloat16)
    acc_ref[...] += jnp.dot(m, d2, preferred_element_type=jnp.float32)

    @pl.when(i == _G - 1)
    def _fin():
        wtab = 1.0 / (s_ref[...] + _EPS) ** 2
        tot = jnp.sum(wtab * acc_ref[...]) * (1.0 / (_N * _T))
        out_ref[...] = tot.reshape(1, 1)


def kernel(yhat, y, b, s):
    b3 = b.astype(jnp.int32).reshape(_G, 1, _BR)
    s2 = s.reshape(_K, 1)
    out = pl.pallas_call(
        _nse_kernel,
        grid=(_G,),
        in_specs=[
            pl.BlockSpec((1, 1, _BR), lambda i: (i, 0, 0)),
            pl.BlockSpec((_K, 1), lambda i: (0, 0)),
            pl.BlockSpec((_BR, _T), lambda i: (i, 0)),
            pl.BlockSpec((_BR, _T), lambda i: (i, 0)),
        ],
        out_specs=pl.BlockSpec((1, 1), lambda i: (0, 0)),
        out_shape=jax.ShapeDtypeStruct((1, 1), jnp.float32),
        scratch_shapes=[pltpu.VMEM((_K, _T), jnp.float32)],
        compiler_params=pltpu.CompilerParams(
            dimension_semantics=("arbitrary",),
        ),
    )(b3, s2, yhat, y)
    return out[0, 0]


# speedup vs baseline: 3.5283x; 1.0559x over previous
import jax
import jax.numpy as jnp
from jax.experimental import pallas as pl
from jax.experimental.pallas import tpu as pltpu

_EPS = 0.1
_N = 16384
_T = 512
_K = 64
_BR = 2048
_G = _N // _BR


def _nse_kernel(b_ref, s_ref, yhat_ref, y_ref, out_ref, acc_ref):
    i = pl.program_id(0)

    @pl.when(i == 0)
    def _init():
        acc_ref[...] = jnp.zeros_like(acc_ref)

    d = yhat_ref[...] - y_ref[...]
    d2 = (d * d).astype(jnp.bfloat16)
    b_row = b_ref[...].reshape(1, _BR)
    kio = jax.lax.broadcasted_iota(jnp.int32, (_K, _BR), 0)
    m = (kio == b_row).astype(jnp.bfloat16)
    acc_ref[...] += jnp.dot(m, d2, preferred_element_type=jnp.float32)

    @pl.when(i == _G - 1)
    def _fin():
        wtab = 1.0 / (s_ref[...] + _EPS) ** 2
        tot = jnp.sum(wtab * acc_ref[...]) * (1.0 / (_N * _T))
        out_ref[...] = tot.reshape(1, 1)


def kernel(yhat, y, b, s):
    b3 = b.astype(jnp.int32).reshape(_G, 1, _BR)
    s2 = s.reshape(_K, 1)
    out = pl.pallas_call(
        _nse_kernel,
        grid=(_G,),
        in_specs=[
            pl.BlockSpec((1, 1, _BR), lambda i: (i, 0, 0)),
            pl.BlockSpec((_K, 1), lambda i: (0, 0)),
            pl.BlockSpec((_BR, _T), lambda i: (i, 0)),
            pl.BlockSpec((_BR, _T), lambda i: (i, 0)),
        ],
        out_specs=pl.BlockSpec((1, 1), lambda i: (0, 0)),
        out_shape=jax.ShapeDtypeStruct((1, 1), jnp.float32),
        scratch_shapes=[pltpu.VMEM((_K, _T), jnp.float32)],
        compiler_params=pltpu.CompilerParams(
            dimension_semantics=("arbitrary",),
        ),
    )(b3, s2, yhat, y)
    return out[0, 0]
